# baseline (speedup 1.0000x reference)
import functools

import jax
import jax.numpy as jnp
from jax import lax
from jax.experimental import pallas as pl
from jax.experimental.pallas import tpu as pltpu
from jax.experimental.pallas import tpu_sc as plsc

NUM_EMB = 1000000
EMB_DIM = 64
MAX_LEN = 200
BATCH = 4096
SEQ = 200

NUM_CORES = 2
NUM_SUBCORES = 16
NUM_WORKERS = NUM_CORES * NUM_SUBCORES
LANES = 16
NPAIR = NUM_EMB // 2
N_CBLK = BATCH // 128
N_OCT = SEQ // 8
N_BLOCKS = N_OCT * N_CBLK
BLK_PER_W = N_BLOCKS // NUM_WORKERS


def _emb_kernel(p_hbm, xt_hbm, pos_hbm, out_hbm,
                pos_v, xv, idx0, idx1, par0, par1, pair0, pair1,
                piece0, piece1, gsem0, gsem1, wsem0, wsem1):
    wid = lax.axis_index("c") * NUM_SUBCORES + lax.axis_index("s")

    pltpu.sync_copy(pos_hbm, pos_v)

    iota16 = lax.iota(jnp.int32, LANES)
    jrows = [iota16 + j * LANES for j in range(4)]
    zero16 = jnp.zeros((LANES,), jnp.int32)

    idx_bufs = (idx0, idx1)
    par_bufs = (par0, par1)
    pair_bufs = (pair0, pair1)
    piece_bufs = (piece0, piece1)
    gsems = (gsem0, gsem1)
    wsems = (wsem0, wsem1)

    def prep_idx(l, slot):
        for g in range(8):
            v = xv[l, pl.ds(g * LANES, LANES)]
            idx_bufs[slot][pl.ds(g * LANES, LANES)] = v >> 1
            par_bufs[slot][pl.ds(g * LANES, LANES)] = (v & 1) << 6

    def issue_gather(slot):
        return pltpu.async_copy(p_hbm.at[idx_bufs[slot]], pair_bufs[slot],
                                gsems[slot])

    def wait_gather(slot):
        pltpu.make_async_copy(p_hbm.at[idx_bufs[slot]], pair_bufs[slot],
                              gsems[slot]).wait()

    def compute(slot, lglob):
        piece = piece_bufs[slot]
        pair = pair_bufs[slot]
        par_v = par_bufs[slot]
        posr = [pos_v[pl.ds(lglob * EMB_DIM + j * LANES, LANES)]
                for j in range(4)]

        @plsc.parallel_loop(0, 128, unroll=4)
        def bbody(b):
            pv = par_v[pl.ds(b, LANES)]
            par = pv[0]
            col = zero16 + b
            for j in range(4):
                val = pair[b, pl.ds(par + j * LANES, LANES)]
                plsc.store_scatter(piece, [jrows[j], col], val + posr[j])

    def issue_write(slot, lglob, c):
        return pltpu.async_copy(
            piece_bufs[slot].at[:, pl.ds(0, 128)],
            out_hbm.at[lglob, :, pl.ds(c * 128, 128)],
            wsems[slot])

    def wait_write(slot):
        pltpu.make_async_copy(piece_bufs[slot].at[:, pl.ds(0, 128)],
                              out_hbm.at[0, :, pl.ds(0, 128)],
                              wsems[slot]).wait()

    def blk_body(bi, carry):
        b = wid * BLK_PER_W + bi
        o = b // N_CBLK
        c = lax.rem(b, N_CBLK)
        pltpu.sync_copy(xt_hbm.at[pl.ds(o * 8, 8), pl.ds(c * 128, 128)], xv)

        prep_idx(0, 0)
        issue_gather(0)
        for l in range(8):
            slot = l % 2
            nxt = 1 - slot
            if l < 7:
                prep_idx(l + 1, nxt)
                issue_gather(nxt)
            wait_gather(slot)

            if l >= 2:
                wait_write(slot)
            else:
                @pl.when(bi > 0)
                def _():
                    wait_write(slot)

            compute(slot, o * 8 + l)
            issue_write(slot, o * 8 + l, c)
        return carry

    lax.fori_loop(0, BLK_PER_W, blk_body, 0)
    wait_write(0)
    wait_write(1)


@jax.jit
def _emb(P, xT, posF):
    mesh = plsc.VectorSubcoreMesh(core_axis_name="c", subcore_axis_name="s")
    f = functools.partial(
        pl.kernel,
        mesh=mesh,
        compiler_params=pltpu.CompilerParams(use_tc_tiling_on_sc=True,
                                             needs_layout_passes=False),
        out_type=jax.ShapeDtypeStruct((SEQ, EMB_DIM, BATCH), jnp.float32),
        scratch_types=[
            pltpu.VMEM((MAX_LEN * EMB_DIM,), jnp.float32),
            pltpu.VMEM((8, 128), jnp.int32),
            pltpu.VMEM((128,), jnp.int32),
            pltpu.VMEM((128,), jnp.int32),
            pltpu.VMEM((144,), jnp.int32),
            pltpu.VMEM((144,), jnp.int32),
            pltpu.VMEM((128, 128), jnp.float32),
            pltpu.VMEM((128, 128), jnp.float32),
            pltpu.VMEM((EMB_DIM, 129), jnp.float32),
            pltpu.VMEM((EMB_DIM, 129), jnp.float32),
            pltpu.SemaphoreType.DMA,
            pltpu.SemaphoreType.DMA,
            pltpu.SemaphoreType.DMA,
            pltpu.SemaphoreType.DMA,
        ],
    )(_emb_kernel)
    return f(P, xT, posF)


def kernel(x, W_in, W_pos):
    P = jnp.reshape(W_in, (NPAIR, 128))
    xT = x.T
    posF = jnp.reshape(W_pos, (MAX_LEN * EMB_DIM,))
    outT = _emb(P, xT, posF)
    return jnp.transpose(outT, (2, 0, 1))


# speedup vs baseline: 1.1276x; 1.1276x over previous
import functools

import jax
import jax.numpy as jnp
from jax import lax
from jax.experimental import pallas as pl
from jax.experimental.pallas import tpu as pltpu
from jax.experimental.pallas import tpu_sc as plsc

NUM_EMB = 1000000
EMB_DIM = 64
MAX_LEN = 200
BATCH = 4096
SEQ = 200

NUM_CORES = 2
NUM_SUBCORES = 16
NUM_WORKERS = NUM_CORES * NUM_SUBCORES
SEQ_PER_W = BATCH // NUM_WORKERS
ROWS_PER_W = SEQ_PER_W * SEQ
LANES = 16
VECS_PER_ROW = EMB_DIM // LANES


def _lookup_kernel(w_hbm, x_hbm, pos_hbm, out_hbm,
                   idx_v, pos_v, buf0, buf1, sem0, sem1):
    wid = lax.axis_index("c") * NUM_SUBCORES + lax.axis_index("s")
    row_base = wid * ROWS_PER_W

    pltpu.sync_copy(x_hbm.at[pl.ds(row_base, ROWS_PER_W)], idx_v)
    pltpu.sync_copy(pos_hbm, pos_v)

    def issue(s, buf, sem):
        idx_slice = idx_v.at[pl.ds(s * SEQ, SEQ)]
        return pltpu.async_copy(w_hbm.at[idx_slice], buf, sem)

    def wait(buf, sem):
        pltpu.make_async_copy(w_hbm.at[idx_v.at[pl.ds(0, SEQ)]], buf, sem).wait()

    def add_pos_and_store(s, buf):
        def body(r, carry):
            for j in range(VECS_PER_ROW):
                plsc.addupdate(buf.at[r, pl.ds(j * LANES, LANES)],
                               pos_v[r, pl.ds(j * LANES, LANES)])
            return carry
        lax.fori_loop(0, SEQ, body, 0)
        pltpu.sync_copy(buf, out_hbm.at[pl.ds(row_base + s * SEQ, SEQ)])

    issue(0, buf0, sem0)

    def loop(g, carry):
        s0 = 2 * g
        s1 = s0 + 1
        issue(s1, buf1, sem1)
        wait(buf0, sem0)
        add_pos_and_store(s0, buf0)

        @pl.when(g < SEQ_PER_W // 2 - 1)
        def _():
            issue(s1 + 1, buf0, sem0)

        wait(buf1, sem1)
        add_pos_and_store(s1, buf1)
        return carry

    lax.fori_loop(0, SEQ_PER_W // 2, loop, 0)


@functools.partial(jax.jit, donate_argnums=())
def _lookup(W_in, x_flat, W_pos):
    mesh = plsc.VectorSubcoreMesh(core_axis_name="c", subcore_axis_name="s")
    f = functools.partial(
        pl.kernel,
        mesh=mesh,
        compiler_params=pltpu.CompilerParams(use_tc_tiling_on_sc=False),
        out_type=jax.ShapeDtypeStruct((BATCH * SEQ, EMB_DIM), jnp.float32),
        scratch_types=[
            pltpu.VMEM((ROWS_PER_W,), jnp.int32),
            pltpu.VMEM((MAX_LEN, EMB_DIM), jnp.float32),
            pltpu.VMEM((SEQ, EMB_DIM), jnp.float32),
            pltpu.VMEM((SEQ, EMB_DIM), jnp.float32),
            pltpu.SemaphoreType.DMA,
            pltpu.SemaphoreType.DMA,
        ],
    )(_lookup_kernel)
    return f(W_in, x_flat, W_pos)


def kernel(x, W_in, W_pos):
    x_flat = x.reshape(-1)
    out = _lookup(W_in, x_flat, W_pos)
    return out.reshape(BATCH, SEQ, EMB_DIM)
